# Initial kernel scaffold; baseline (speedup 1.0000x reference)
import jax
import jax.numpy as jnp
from jax.experimental import pallas as pl


def kernel(x1, edge_index1, x2, edge_index2, W1, b1, prelu_a, W2, b2):
    raise NotImplementedError("write your pallas kernel here")


# speedup vs baseline: 1.7335x; 1.7335x over previous
import functools

import jax
import jax.numpy as jnp
from jax.experimental import pallas as pl


def _mm(x, wcat, bcat):
    n, d = x.shape
    k = wcat.shape[1]
    bn = 1000

    def body(x_ref, w_ref, b_ref, o_ref):
        o_ref[...] = (
            jnp.dot(x_ref[...], w_ref[...], preferred_element_type=jnp.float32)
            + b_ref[...]
        )

    return pl.pallas_call(
        body,
        grid=(n // bn,),
        in_specs=[
            pl.BlockSpec((bn, d), lambda i: (i, 0)),
            pl.BlockSpec((d, k), lambda i: (0, 0)),
            pl.BlockSpec((1, k), lambda i: (0, 0)),
        ],
        out_specs=pl.BlockSpec((bn, k), lambda i: (i, 0)),
        out_shape=jax.ShapeDtypeStruct((n, k), jnp.float32),
    )(x, wcat, bcat.reshape(1, k))


def _edge_conv(x, src, dst, wcat, bcat):
    n, d = x.shape
    pq = _mm(x, wcat, bcat)
    p, q = pq[:, :d], pq[:, d:]
    s = jax.ops.segment_max(q[src], dst, num_segments=n)
    return jnp.where(jnp.isfinite(s), p + s, 0.0)


def _embed(x, src, dst, wc1, bc1, prelu_a, wc2, bc2):
    h = _edge_conv(x, src, dst, wc1, bc1)
    h = jnp.where(h >= 0, h, prelu_a * h)
    return _edge_conv(h, src, dst, wc2, bc2)


@jax.jit
def kernel(x1, edge_index1, x2, edge_index2, W1, b1, prelu_a, W2, b2):
    d = x1.shape[1]
    dh = W1.shape[1]
    wc1 = jnp.concatenate([W1[:d] - W1[d:], W1[d:]], axis=1)
    bc1 = jnp.concatenate([b1, jnp.zeros_like(b1)])
    wc2 = jnp.concatenate([W2[:dh] - W2[dh:], W2[dh:]], axis=1)
    bc2 = jnp.concatenate([b2, jnp.zeros_like(b2)])
    out1 = _embed(x1, edge_index1[0], edge_index1[1], wc1, bc1, prelu_a, wc2, bc2)
    out2 = _embed(x2, edge_index2[0], edge_index2[1], wc1, bc1, prelu_a, wc2, bc2)
    return (out1, out2)
